# baseline (speedup 1.0000x reference)
import functools

import jax
import jax.numpy as jnp
from jax import lax
from jax.experimental import pallas as pl
from jax.experimental.pallas import tpu as pltpu
from jax.experimental.pallas import tpu_sc as plsc

NC = 2
NS = 16
NW = NC * NS
CH = 128


def _make_sc_gather(D, B, n_tables, ch):
    assert B % ch == 0
    n_chunks = B // ch
    n_rounds = n_chunks // NW
    rem = n_chunks % NW
    mesh = plsc.VectorSubcoreMesh(core_axis_name="c", subcore_axis_name="s",
                                  num_cores=NC, num_subcores=NS)

    scratch = []
    for t in range(n_tables):
        scratch.append(pltpu.VMEM((ch,), jnp.int32))
        scratch.append(pltpu.VMEM((ch, D[t]), jnp.float32))
        scratch.append(pltpu.SemaphoreType.DMA)

    @functools.partial(
        pl.kernel,
        out_type=tuple(jax.ShapeDtypeStruct((B, D[t]), jnp.float32)
                       for t in range(n_tables)),
        mesh=mesh,
        scratch_types=scratch,
    )
    def k(*refs):
        tables = refs[:n_tables]
        idxs = refs[n_tables:2 * n_tables]
        outs = refs[2 * n_tables:3 * n_tables]
        scr = refs[3 * n_tables:]
        wid = lax.axis_index("s") * NC + lax.axis_index("c")

        def do_chunk(c):
            for t in range(n_tables):
                idx_v, rows_v, sem = scr[3 * t], scr[3 * t + 1], scr[3 * t + 2]
                pltpu.sync_copy(idxs[t].at[pl.ds(c * ch, ch)], idx_v)
                pltpu.async_copy(tables[t].at[idx_v], rows_v, sem).wait()
                pltpu.sync_copy(rows_v, outs[t].at[pl.ds(c * ch, ch)])

        if n_rounds > 0:
            def body(r, carry):
                do_chunk(r * NW + wid)
                return carry
            lax.fori_loop(0, n_rounds, body, 0)
        if rem > 0:
            @pl.when(wid < rem)
            def _():
                do_chunk(n_rounds * NW + wid)

    return k


def _edge_tc(e_perm, xg, row_s3, W1a, W1b, b1, W2, b2, *, H, EB, NACC, WIN):
    n_eblk = e_perm.shape[0] // EB
    De = e_perm.shape[1]

    def body(e_ref, xg_ref, row_ref, W1a_ref, W1b_ref, b1_ref, W2_ref, b2_ref,
             acc_ref, cnt_ref):
        i = pl.program_id(0)

        @pl.when(i == 0)
        def _init():
            acc_ref[...] = jnp.zeros_like(acc_ref)
            cnt_ref[...] = jnp.zeros_like(cnt_ref)

        e_blk = e_ref[...].astype(jnp.bfloat16)
        xg_blk = xg_ref[...]
        rows = row_ref[0]
        h = jnp.dot(e_blk, W1b_ref[...], preferred_element_type=jnp.float32)
        h += jnp.dot(xg_blk, W1a_ref[...], preferred_element_type=jnp.float32)
        h += b1_ref[...]
        h = jnp.maximum(h, 0.0)
        h = jnp.dot(h.astype(jnp.bfloat16), W2_ref[...],
                    preferred_element_type=jnp.float32)
        h += b2_ref[...]
        h = jnp.maximum(h, 0.0)
        hb = h.astype(jnp.bfloat16)

        lo = row_ref[0, 0, 0]
        hi = row_ref[0, 0, EB - 1]
        base = (lo // WIN) * WIN
        nwin = (hi - base) // WIN + 1

        def win(k, carry):
            wbase = base + k * WIN
            rel = rows - wbase
            jj = lax.broadcasted_iota(jnp.int32, (WIN, EB), 0)
            oh = (jj == rel).astype(jnp.bfloat16)
            part = jnp.dot(oh, hb, preferred_element_type=jnp.float32)
            pcnt = jnp.sum(oh.astype(jnp.float32), axis=1, keepdims=True)
            acc_ref[pl.ds(wbase, WIN), :] += part
            cnt_ref[pl.ds(wbase, WIN), :] += pcnt
            return carry

        lax.fori_loop(0, nwin, win, 0)

    def eix(i):
        return (i, 0)

    def eix3(i):
        return (i, 0, 0)

    def cix(i):
        return (0, 0)

    return pl.pallas_call(
        body,
        grid=(n_eblk,),
        in_specs=[
            pl.BlockSpec((EB, De), eix),
            pl.BlockSpec((EB, 128), eix),
            pl.BlockSpec((1, 1, EB), eix3),
            pl.BlockSpec((128, H), cix),
            pl.BlockSpec((De, H), cix),
            pl.BlockSpec((1, H), cix),
            pl.BlockSpec((H, H), cix),
            pl.BlockSpec((1, H), cix),
        ],
        out_specs=(pl.BlockSpec((NACC, H), cix),
                   pl.BlockSpec((NACC, 128), cix)),
        out_shape=(jax.ShapeDtypeStruct((NACC, H), jnp.float32),
                   jax.ShapeDtypeStruct((NACC, 128), jnp.float32)),
    )(e_perm, xg, row_s3, W1a, W1b, b1, W2, b2)


def _node_tc(accs, cnts, x_pad, u, batch2, W3a, W3b, W3c, b3, W4p, b4p,
             *, N, H, NB):
    K = len(accs)
    G, Du = u.shape
    n_nblk = N // NB

    def body(*refs):
        acc_refs = refs[:K]
        cnt_refs = refs[K:2 * K]
        (x_ref, u_ref, batch_ref, W3a_ref, W3b_ref, W3c_ref, b3_ref,
         W4_ref, b4_ref, out_ref) = refs[2 * K:]
        sums = acc_refs[0][...]
        cnt = cnt_refs[0][:, 0:1]
        for t in range(1, K):
            sums += acc_refs[t][...]
            cnt += cnt_refs[t][:, 0:1]
        mean = sums / jnp.maximum(cnt, 1.0)
        t = jnp.dot(x_ref[...], W3a_ref[...], preferred_element_type=jnp.float32)
        t += jnp.dot(mean, W3b_ref[...], preferred_element_type=jnp.float32)
        bb = batch_ref[...]
        gg = lax.broadcasted_iota(jnp.int32, (NB, G), 1)
        ohu = (gg == bb).astype(jnp.float32)
        ug_blk = jnp.dot(ohu, u_ref[...], preferred_element_type=jnp.float32)
        t += jnp.dot(ug_blk, W3c_ref[...], preferred_element_type=jnp.float32)
        t += b3_ref[...]
        t = jnp.maximum(t, 0.0)
        out_ref[...] = (jnp.dot(t, W4_ref[...], preferred_element_type=jnp.float32)
                        + b4_ref[...])

    def nix(i):
        return (i, 0)

    def cix(i):
        return (0, 0)

    in_specs = ([pl.BlockSpec((NB, H), nix)] * K
                + [pl.BlockSpec((NB, 128), nix)] * K
                + [
                    pl.BlockSpec((NB, 128), nix),
                    pl.BlockSpec((G, Du), cix),
                    pl.BlockSpec((NB, 1), nix),
                    pl.BlockSpec((128, H), cix),
                    pl.BlockSpec((H, H), cix),
                    pl.BlockSpec((Du, H), cix),
                    pl.BlockSpec((1, H), cix),
                    pl.BlockSpec((H, 128), cix),
                    pl.BlockSpec((1, 128), cix),
                ])

    return pl.pallas_call(
        body,
        grid=(n_nblk,),
        in_specs=in_specs,
        out_specs=pl.BlockSpec((NB, 128), nix),
        out_shape=jax.ShapeDtypeStruct((N, 128), jnp.float32),
    )(*accs, *cnts, x_pad, u, batch2, W3a, W3b, W3c, b3, W4p, b4p)


def kernel(x, edge_index, edge_attr, u, batch, W1, b1, W2, b2, W3, b3, W4, b4):
    N, Dn = x.shape
    E, De = edge_attr.shape
    G, Du = u.shape
    H = W2.shape[0]
    D_OUT = W4.shape[1]

    EB = 1600
    NB = 1000
    WIN = 128
    CHUNK_BLOCKS = (8, 16, 20, 20, 20, 16)

    row = edge_index[0].astype(jnp.int32)
    col = edge_index[1].astype(jnp.int32)

    x_pad = jnp.zeros((N, 128), jnp.float32).at[:, :Dn].set(x)
    W1a = jnp.zeros((128, H), jnp.float32).at[:Dn].set(W1[:Dn])
    W1b = W1[Dn:].astype(jnp.bfloat16)
    W3a = jnp.zeros((128, H), jnp.float32).at[:Dn].set(W3[:Dn])
    W3b = W3[Dn:Dn + H]
    W3c = W3[Dn + H:]
    W4p = jnp.zeros((H, 128), jnp.float32).at[:, :D_OUT].set(W4)
    b4p = jnp.zeros((1, 128), jnp.float32).at[0, :D_OUT].set(b4)
    batch2 = batch.astype(jnp.int32).reshape(N, 1)
    W2bf = W2.astype(jnp.bfloat16)
    b1r = b1.reshape(1, H)
    b2r = b2.reshape(1, H)
    b3r = b3.reshape(1, H)

    NACC = ((N + WIN) // WIN + 1) * WIN
    assert sum(CHUNK_BLOCKS) * EB == E

    gather_kernels = {}
    accs, cnts = [], []
    base = 0
    for nblk in CHUNK_BLOCKS:
        epc = nblk * EB
        assert epc < (1 << 15) and epc % CH == 0
        if epc not in gather_kernels:
            gather_kernels[epc] = _make_sc_gather((De, 128), epc, 2, CH)
        sl = slice(base, base + epc)
        keys = lax.sort(row[sl] * (1 << 15) + lax.iota(jnp.int32, epc))
        row_c = keys >> 15
        perm_local = keys & ((1 << 15) - 1)
        perm_c = perm_local + base
        col_c = jnp.take(col[sl], perm_local)
        e_c, xg_c = gather_kernels[epc](edge_attr, x_pad, perm_c, col_c)
        a_c, n_c = _edge_tc(e_c, xg_c, row_c.reshape(nblk, 1, EB),
                            W1a, W1b, b1r, W2bf, b2r,
                            H=H, EB=EB, NACC=NACC, WIN=WIN)
        accs.append(a_c)
        cnts.append(n_c)
        base += epc

    out_pad = _node_tc(accs, cnts, x_pad, u, batch2,
                       W3a, W3b, W3c, b3r, W4p, b4p, N=N, H=H, NB=NB)
    return out_pad[:, :D_OUT]


# speedup vs baseline: 1.0788x; 1.0788x over previous
import functools

import jax
import jax.numpy as jnp
from jax import lax
from jax.experimental import pallas as pl
from jax.experimental.pallas import tpu as pltpu
from jax.experimental.pallas import tpu_sc as plsc

NC = 2
NS = 16
NW = NC * NS
CH = 128


def _make_sc_gather(D, B, n_tables, ch):
    assert B % ch == 0
    n_chunks = B // ch
    n_rounds = n_chunks // NW
    rem = n_chunks % NW
    mesh = plsc.VectorSubcoreMesh(core_axis_name="c", subcore_axis_name="s",
                                  num_cores=NC, num_subcores=NS)

    scratch = []
    for t in range(n_tables):
        scratch.append(pltpu.VMEM((ch,), jnp.int32))
        scratch.append(pltpu.VMEM((ch, D[t]), jnp.float32))
        scratch.append(pltpu.SemaphoreType.DMA)

    @functools.partial(
        pl.kernel,
        out_type=tuple(jax.ShapeDtypeStruct((B, D[t]), jnp.float32)
                       for t in range(n_tables)),
        mesh=mesh,
        scratch_types=scratch,
    )
    def k(*refs):
        tables = refs[:n_tables]
        idxs = refs[n_tables:2 * n_tables]
        outs = refs[2 * n_tables:3 * n_tables]
        scr = refs[3 * n_tables:]
        wid = lax.axis_index("s") * NC + lax.axis_index("c")

        def do_chunk(c):
            for t in range(n_tables):
                idx_v, rows_v, sem = scr[3 * t], scr[3 * t + 1], scr[3 * t + 2]
                pltpu.sync_copy(idxs[t].at[pl.ds(c * ch, ch)], idx_v)
                pltpu.async_copy(tables[t].at[idx_v], rows_v, sem).wait()
                pltpu.sync_copy(rows_v, outs[t].at[pl.ds(c * ch, ch)])

        if n_rounds > 0:
            def body(r, carry):
                do_chunk(r * NW + wid)
                return carry
            lax.fori_loop(0, n_rounds, body, 0)
        if rem > 0:
            @pl.when(wid < rem)
            def _():
                do_chunk(n_rounds * NW + wid)

    return k


def _edge_tc(e_perm, xg, row_s3, W1a, W1b, b1, W2, b2, *, H, EB, NACC, WIN):
    n_eblk = e_perm.shape[0] // EB
    De = e_perm.shape[1]

    def body(e_ref, xg_ref, row_ref, W1a_ref, W1b_ref, b1_ref, W2_ref, b2_ref,
             acc_ref, cnt_ref):
        i = pl.program_id(0)

        @pl.when(i == 0)
        def _init():
            acc_ref[...] = jnp.zeros_like(acc_ref)
            cnt_ref[...] = jnp.zeros_like(cnt_ref)

        e_blk = e_ref[...].astype(jnp.bfloat16)
        xg_blk = xg_ref[...]
        rows = row_ref[0]
        h = jnp.dot(e_blk, W1b_ref[...], preferred_element_type=jnp.float32)
        h += jnp.dot(xg_blk, W1a_ref[...], preferred_element_type=jnp.float32)
        h += b1_ref[...]
        h = jnp.maximum(h, 0.0)
        h = jnp.dot(h.astype(jnp.bfloat16), W2_ref[...],
                    preferred_element_type=jnp.float32)
        h += b2_ref[...]
        h = jnp.maximum(h, 0.0)
        hb = h.astype(jnp.bfloat16)

        lo = row_ref[0, 0, 0]
        hi = row_ref[0, 0, EB - 1]
        base = (lo // WIN) * WIN
        nwin = (hi - base) // WIN + 1

        def win(k, carry):
            wbase = base + k * WIN
            rel = rows - wbase
            jj = lax.broadcasted_iota(jnp.int32, (WIN, EB), 0)
            oh = (jj == rel).astype(jnp.bfloat16)
            part = jnp.dot(oh, hb, preferred_element_type=jnp.float32)
            pcnt = jnp.sum(oh.astype(jnp.float32), axis=1, keepdims=True)
            acc_ref[pl.ds(wbase, WIN), :] += part
            cnt_ref[pl.ds(wbase, WIN), :] += pcnt
            return carry

        lax.fori_loop(0, nwin, win, 0)

    def eix(i):
        return (i, 0)

    def eix3(i):
        return (i, 0, 0)

    def cix(i):
        return (0, 0)

    return pl.pallas_call(
        body,
        grid=(n_eblk,),
        in_specs=[
            pl.BlockSpec((EB, De), eix),
            pl.BlockSpec((EB, 128), eix),
            pl.BlockSpec((1, 1, EB), eix3),
            pl.BlockSpec((128, H), cix),
            pl.BlockSpec((De, H), cix),
            pl.BlockSpec((1, H), cix),
            pl.BlockSpec((H, H), cix),
            pl.BlockSpec((1, H), cix),
        ],
        out_specs=(pl.BlockSpec((NACC, H), cix),
                   pl.BlockSpec((NACC, 128), cix)),
        out_shape=(jax.ShapeDtypeStruct((NACC, H), jnp.float32),
                   jax.ShapeDtypeStruct((NACC, 128), jnp.float32)),
    )(e_perm, xg, row_s3, W1a, W1b, b1, W2, b2)


def _node_tc(accs, cnts, x_pad, u, batch2, W3a, W3b, W3c, b3, W4p, b4p,
             *, N, H, NB):
    K = len(accs)
    G, Du = u.shape
    n_nblk = N // NB

    def body(*refs):
        acc_refs = refs[:K]
        cnt_refs = refs[K:2 * K]
        (x_ref, u_ref, batch_ref, W3a_ref, W3b_ref, W3c_ref, b3_ref,
         W4_ref, b4_ref, out_ref) = refs[2 * K:]
        sums = acc_refs[0][...]
        cnt = cnt_refs[0][:, 0:1]
        for t in range(1, K):
            sums += acc_refs[t][...]
            cnt += cnt_refs[t][:, 0:1]
        mean = sums / jnp.maximum(cnt, 1.0)
        t = jnp.dot(x_ref[...], W3a_ref[...], preferred_element_type=jnp.float32)
        t += jnp.dot(mean, W3b_ref[...], preferred_element_type=jnp.float32)
        bb = batch_ref[...]
        gg = lax.broadcasted_iota(jnp.int32, (NB, G), 1)
        ohu = (gg == bb).astype(jnp.float32)
        ug_blk = jnp.dot(ohu, u_ref[...], preferred_element_type=jnp.float32)
        t += jnp.dot(ug_blk, W3c_ref[...], preferred_element_type=jnp.float32)
        t += b3_ref[...]
        t = jnp.maximum(t, 0.0)
        out_ref[...] = (jnp.dot(t, W4_ref[...], preferred_element_type=jnp.float32)
                        + b4_ref[...])

    def nix(i):
        return (i, 0)

    def cix(i):
        return (0, 0)

    in_specs = ([pl.BlockSpec((NB, H), nix)] * K
                + [pl.BlockSpec((NB, 128), nix)] * K
                + [
                    pl.BlockSpec((NB, 128), nix),
                    pl.BlockSpec((G, Du), cix),
                    pl.BlockSpec((NB, 1), nix),
                    pl.BlockSpec((128, H), cix),
                    pl.BlockSpec((H, H), cix),
                    pl.BlockSpec((Du, H), cix),
                    pl.BlockSpec((1, H), cix),
                    pl.BlockSpec((H, 128), cix),
                    pl.BlockSpec((1, 128), cix),
                ])

    return pl.pallas_call(
        body,
        grid=(n_nblk,),
        in_specs=in_specs,
        out_specs=pl.BlockSpec((NB, 128), nix),
        out_shape=jax.ShapeDtypeStruct((N, 128), jnp.float32),
    )(*accs, *cnts, x_pad, u, batch2, W3a, W3b, W3c, b3, W4p, b4p)


def kernel(x, edge_index, edge_attr, u, batch, W1, b1, W2, b2, W3, b3, W4, b4):
    N, Dn = x.shape
    E, De = edge_attr.shape
    G, Du = u.shape
    H = W2.shape[0]
    D_OUT = W4.shape[1]

    EB = 1600
    NB = 1000
    WIN = 128
    CHUNK_BLOCKS = (20, 20, 20, 20, 20)

    row = edge_index[0].astype(jnp.int32)
    col = edge_index[1].astype(jnp.int32)

    x_pad = jnp.zeros((N, 128), jnp.float32).at[:, :Dn].set(x)
    W1a = jnp.zeros((128, H), jnp.float32).at[:Dn].set(W1[:Dn])
    W1b = W1[Dn:].astype(jnp.bfloat16)
    W3a = jnp.zeros((128, H), jnp.float32).at[:Dn].set(W3[:Dn])
    W3b = W3[Dn:Dn + H]
    W3c = W3[Dn + H:]
    W4p = jnp.zeros((H, 128), jnp.float32).at[:, :D_OUT].set(W4)
    b4p = jnp.zeros((1, 128), jnp.float32).at[0, :D_OUT].set(b4)
    batch2 = batch.astype(jnp.int32).reshape(N, 1)
    W2bf = W2.astype(jnp.bfloat16)
    b1r = b1.reshape(1, H)
    b2r = b2.reshape(1, H)
    b3r = b3.reshape(1, H)

    NACC = ((N + WIN) // WIN + 1) * WIN
    assert sum(CHUNK_BLOCKS) * EB == E

    gather_kernels = {}
    accs, cnts = [], []
    base = 0
    for nblk in CHUNK_BLOCKS:
        epc = nblk * EB
        assert epc < (1 << 15) and epc % CH == 0
        if epc not in gather_kernels:
            gather_kernels[epc] = _make_sc_gather((De, 128), epc, 2, CH)
        sl = slice(base, base + epc)
        keys = lax.sort(row[sl] * (1 << 15) + lax.iota(jnp.int32, epc))
        row_c = keys >> 15
        perm_local = keys & ((1 << 15) - 1)
        perm_c = perm_local + base
        col_c = jnp.take(col[sl], perm_local)
        e_c, xg_c = gather_kernels[epc](edge_attr, x_pad, perm_c, col_c)
        a_c, n_c = _edge_tc(e_c, xg_c, row_c.reshape(nblk, 1, EB),
                            W1a, W1b, b1r, W2bf, b2r,
                            H=H, EB=EB, NACC=NACC, WIN=WIN)
        accs.append(a_c)
        cnts.append(n_c)
        base += epc

    out_pad = _node_tc(accs, cnts, x_pad, u, batch2,
                       W3a, W3b, W3c, b3r, W4p, b4p, N=N, H=H, NB=NB)
    return out_pad[:, :D_OUT]


# speedup vs baseline: 1.1020x; 1.0215x over previous
import functools

import jax
import jax.numpy as jnp
from jax import lax
from jax.experimental import pallas as pl
from jax.experimental.pallas import tpu as pltpu
from jax.experimental.pallas import tpu_sc as plsc

NC = 2
NS = 16
NW = NC * NS
CH = 128


def _make_sc_gather(D, B, n_tables, ch):
    assert B % ch == 0
    n_chunks = B // ch
    n_rounds = n_chunks // NW
    rem = n_chunks % NW
    mesh = plsc.VectorSubcoreMesh(core_axis_name="c", subcore_axis_name="s",
                                  num_cores=NC, num_subcores=NS)

    scratch = []
    for t in range(n_tables):
        scratch.append(pltpu.VMEM((ch,), jnp.int32))
        scratch.append(pltpu.VMEM((ch, D[t]), jnp.float32))
        scratch.append(pltpu.SemaphoreType.DMA)

    @functools.partial(
        pl.kernel,
        out_type=tuple(jax.ShapeDtypeStruct((B, D[t]), jnp.float32)
                       for t in range(n_tables)),
        mesh=mesh,
        scratch_types=scratch,
    )
    def k(*refs):
        tables = refs[:n_tables]
        idxs = refs[n_tables:2 * n_tables]
        outs = refs[2 * n_tables:3 * n_tables]
        scr = refs[3 * n_tables:]
        wid = lax.axis_index("s") * NC + lax.axis_index("c")

        def do_chunk(c):
            for t in range(n_tables):
                idx_v, rows_v, sem = scr[3 * t], scr[3 * t + 1], scr[3 * t + 2]
                pltpu.sync_copy(idxs[t].at[pl.ds(c * ch, ch)], idx_v)
                pltpu.async_copy(tables[t].at[idx_v], rows_v, sem).wait()
                pltpu.sync_copy(rows_v, outs[t].at[pl.ds(c * ch, ch)])

        if n_rounds > 0:
            def body(r, carry):
                do_chunk(r * NW + wid)
                return carry
            lax.fori_loop(0, n_rounds, body, 0)
        if rem > 0:
            @pl.when(wid < rem)
            def _():
                do_chunk(n_rounds * NW + wid)

    return k


def _edge_tc(e_perm, xg, row_s3, W1a, W1b, b1, W2, b2, *, H, EB, NACC, WIN):
    n_eblk = e_perm.shape[0] // EB
    De = e_perm.shape[1]

    def body(e_ref, xg_ref, row_ref, W1a_ref, W1b_ref, b1_ref, W2_ref, b2_ref,
             acc_ref, cnt_ref):
        i = pl.program_id(0)

        @pl.when(i == 0)
        def _init():
            acc_ref[...] = jnp.zeros_like(acc_ref)
            cnt_ref[...] = jnp.zeros_like(cnt_ref)

        e_blk = e_ref[...].astype(jnp.bfloat16)
        xg_blk = xg_ref[...]
        rows = row_ref[0]
        h = jnp.dot(e_blk, W1b_ref[...], preferred_element_type=jnp.float32)
        h += jnp.dot(xg_blk, W1a_ref[...], preferred_element_type=jnp.float32)
        h += b1_ref[...]
        h = jnp.maximum(h, 0.0)
        h = jnp.dot(h.astype(jnp.bfloat16), W2_ref[...],
                    preferred_element_type=jnp.float32)
        h += b2_ref[...]
        h = jnp.maximum(h, 0.0)
        hb = h.astype(jnp.bfloat16)

        lo = row_ref[0, 0, 0]
        hi = row_ref[0, 0, EB - 1]
        base = (lo // WIN) * WIN
        nwin = (hi - base) // WIN + 1

        def win(k, carry):
            wbase = base + k * WIN
            rel = rows - wbase
            jj = lax.broadcasted_iota(jnp.int32, (WIN, EB), 0)
            oh = (jj == rel).astype(jnp.bfloat16)
            part = jnp.dot(oh, hb, preferred_element_type=jnp.float32)
            pcnt = jnp.sum(oh.astype(jnp.float32), axis=1, keepdims=True)
            acc_ref[pl.ds(wbase, WIN), :] += part
            cnt_ref[pl.ds(wbase, WIN), :] += pcnt
            return carry

        lax.fori_loop(0, nwin, win, 0)

    def eix(i):
        return (i, 0)

    def eix3(i):
        return (i, 0, 0)

    def cix(i):
        return (0, 0)

    return pl.pallas_call(
        body,
        grid=(n_eblk,),
        in_specs=[
            pl.BlockSpec((EB, De), eix),
            pl.BlockSpec((EB, 128), eix),
            pl.BlockSpec((1, 1, EB), eix3),
            pl.BlockSpec((128, H), cix),
            pl.BlockSpec((De, H), cix),
            pl.BlockSpec((1, H), cix),
            pl.BlockSpec((H, H), cix),
            pl.BlockSpec((1, H), cix),
        ],
        out_specs=(pl.BlockSpec((NACC, H), cix),
                   pl.BlockSpec((NACC, 128), cix)),
        out_shape=(jax.ShapeDtypeStruct((NACC, H), jnp.float32),
                   jax.ShapeDtypeStruct((NACC, 128), jnp.float32)),
    )(e_perm, xg, row_s3, W1a, W1b, b1, W2, b2)


def _node_tc(accs, cnts, x_pad, u, batch2, W3a, W3b, W3c, b3, W4p, b4p,
             *, N, H, NB):
    K = len(accs)
    G, Du = u.shape
    n_nblk = N // NB

    def body(*refs):
        acc_refs = refs[:K]
        cnt_refs = refs[K:2 * K]
        (x_ref, u_ref, batch_ref, W3a_ref, W3b_ref, W3c_ref, b3_ref,
         W4_ref, b4_ref, out_ref) = refs[2 * K:]
        sums = acc_refs[0][...]
        cnt = cnt_refs[0][:, 0:1]
        for t in range(1, K):
            sums += acc_refs[t][...]
            cnt += cnt_refs[t][:, 0:1]
        mean = sums / jnp.maximum(cnt, 1.0)
        t = jnp.dot(x_ref[...], W3a_ref[...], preferred_element_type=jnp.float32)
        t += jnp.dot(mean, W3b_ref[...], preferred_element_type=jnp.float32)
        bb = batch_ref[...]
        gg = lax.broadcasted_iota(jnp.int32, (NB, G), 1)
        ohu = (gg == bb).astype(jnp.float32)
        ug_blk = jnp.dot(ohu, u_ref[...], preferred_element_type=jnp.float32)
        t += jnp.dot(ug_blk, W3c_ref[...], preferred_element_type=jnp.float32)
        t += b3_ref[...]
        t = jnp.maximum(t, 0.0)
        out_ref[...] = (jnp.dot(t, W4_ref[...], preferred_element_type=jnp.float32)
                        + b4_ref[...])

    def nix(i):
        return (i, 0)

    def cix(i):
        return (0, 0)

    in_specs = ([pl.BlockSpec((NB, H), nix)] * K
                + [pl.BlockSpec((NB, 128), nix)] * K
                + [
                    pl.BlockSpec((NB, 128), nix),
                    pl.BlockSpec((G, Du), cix),
                    pl.BlockSpec((NB, 1), nix),
                    pl.BlockSpec((128, H), cix),
                    pl.BlockSpec((H, H), cix),
                    pl.BlockSpec((Du, H), cix),
                    pl.BlockSpec((1, H), cix),
                    pl.BlockSpec((H, 128), cix),
                    pl.BlockSpec((1, 128), cix),
                ])

    return pl.pallas_call(
        body,
        grid=(n_nblk,),
        in_specs=in_specs,
        out_specs=pl.BlockSpec((NB, 128), nix),
        out_shape=jax.ShapeDtypeStruct((N, 128), jnp.float32),
    )(*accs, *cnts, x_pad, u, batch2, W3a, W3b, W3c, b3, W4p, b4p)


def kernel(x, edge_index, edge_attr, u, batch, W1, b1, W2, b2, W3, b3, W4, b4):
    N, Dn = x.shape
    E, De = edge_attr.shape
    G, Du = u.shape
    H = W2.shape[0]
    D_OUT = W4.shape[1]

    EB = 1280
    NB = 1000
    WIN = 128
    CHUNK_BLOCKS = (25, 25, 25, 25, 25)

    row = edge_index[0].astype(jnp.int32)
    col = edge_index[1].astype(jnp.int32)

    x_pad = jnp.zeros((N, 128), jnp.float32).at[:, :Dn].set(x)
    W1a = jnp.zeros((128, H), jnp.float32).at[:Dn].set(W1[:Dn])
    W1b = W1[Dn:].astype(jnp.bfloat16)
    W3a = jnp.zeros((128, H), jnp.float32).at[:Dn].set(W3[:Dn])
    W3b = W3[Dn:Dn + H]
    W3c = W3[Dn + H:]
    W4p = jnp.zeros((H, 128), jnp.float32).at[:, :D_OUT].set(W4)
    b4p = jnp.zeros((1, 128), jnp.float32).at[0, :D_OUT].set(b4)
    batch2 = batch.astype(jnp.int32).reshape(N, 1)
    W2bf = W2.astype(jnp.bfloat16)
    b1r = b1.reshape(1, H)
    b2r = b2.reshape(1, H)
    b3r = b3.reshape(1, H)

    NACC = ((N + WIN) // WIN + 1) * WIN
    assert sum(CHUNK_BLOCKS) * EB == E

    gather_kernels = {}
    accs, cnts = [], []
    base = 0
    for nblk in CHUNK_BLOCKS:
        epc = nblk * EB
        assert epc < (1 << 15) and epc % CH == 0
        if epc not in gather_kernels:
            gather_kernels[epc] = _make_sc_gather((De, 128), epc, 2, CH)
        sl = slice(base, base + epc)
        keys = lax.sort(row[sl] * (1 << 15) + lax.iota(jnp.int32, epc))
        row_c = keys >> 15
        perm_local = keys & ((1 << 15) - 1)
        perm_c = perm_local + base
        col_c = jnp.take(col[sl], perm_local)
        e_c, xg_c = gather_kernels[epc](edge_attr, x_pad, perm_c, col_c)
        a_c, n_c = _edge_tc(e_c, xg_c, row_c.reshape(nblk, 1, EB),
                            W1a, W1b, b1r, W2bf, b2r,
                            H=H, EB=EB, NACC=NACC, WIN=WIN)
        accs.append(a_c)
        cnts.append(n_c)
        base += epc

    out_pad = _node_tc(accs, cnts, x_pad, u, batch2,
                       W3a, W3b, W3c, b3r, W4p, b4p, N=N, H=H, NB=NB)
    return out_pad[:, :D_OUT]
